# baseline (speedup 1.0000x reference)
import functools

import jax
import jax.numpy as jnp
from jax import lax
from jax.experimental import pallas as pl
from jax.experimental.pallas import tpu as pltpu
from jax.experimental.pallas import tpu_sc as plsc

N = 10000
E = 160000
D_IN = 256
H1 = 512
H2 = 256

LANE = 128
NS = 16
NC = 2
EPT = E // NS
G = 312
NG = EPT // G
GT = EPT - NG * G
GA = 112
NFULL = EPT // GA
AT = EPT - NFULL * GA
NP = 10112
RPT = NP // NS

_mesh = lambda: plsc.VectorSubcoreMesh(
    core_axis_name="c", subcore_axis_name="s", num_cores=NC, num_subcores=NS)


def _sc_degree(eidx_flat, ones128, zeros128):
    def body(eidx_hbm, ones_hbm, zeros_hbm, deg_hbm, idx_v, ones_v, dsem, acc):
        ci = lax.axis_index("c")
        s = lax.axis_index("s")
        pltpu.sync_copy(eidx_hbm.at[pl.ds(ci * E + s * EPT, EPT)], idx_v)
        pltpu.sync_copy(ones_hbm, ones_v)
        pltpu.sync_copy(zeros_hbm.at[pl.ds(s * RPT, RPT)],
                        acc.at[pl.ds(s * RPT, RPT)])
        plsc.subcore_barrier()

        W = 8

        def grp(g, carry):
            pltpu.async_copy(ones_v, acc.at[idx_v.at[pl.ds(g * G, G)]], dsem,
                             add=True)

            @pl.when(g >= W)
            def _():
                pltpu.make_async_copy(
                    ones_v, acc.at[idx_v.at[pl.ds((g - W) * G, G)]], dsem
                ).wait()

            return carry

        lax.fori_loop(0, NG, grp, 0)
        pltpu.async_copy(ones_v.at[pl.ds(0, GT)],
                         acc.at[idx_v.at[pl.ds(NG * G, GT)]], dsem, add=True)

        def drain(g, carry):
            pltpu.make_async_copy(
                ones_v, acc.at[idx_v.at[pl.ds(g * G, G)]], dsem).wait()
            return carry

        lax.fori_loop(NG - W, NG, drain, 0)
        pltpu.make_async_copy(ones_v.at[pl.ds(0, GT)],
                              acc.at[idx_v.at[pl.ds(NG * G, GT)]], dsem).wait()
        plsc.subcore_barrier()
        pltpu.sync_copy(acc.at[pl.ds(s * RPT, RPT)],
                        deg_hbm.at[ci, pl.ds(s * RPT, RPT)])

    return pl.kernel(
        body,
        out_type=jax.ShapeDtypeStruct((NC, NP, LANE), jnp.float32),
        mesh=_mesh(),
        scratch_types=[
            pltpu.VMEM((EPT,), jnp.int32),
            pltpu.VMEM((G, LANE), jnp.float32),
            pltpu.SemaphoreType.DMA,
            pltpu.VMEM_SHARED((NP, LANE), jnp.float32),
        ],
    )(eidx_flat, ones128, zeros128)


def _sc_aggregate(hc_flat, src, dst, zeros128, n_chunks):
    cpc = n_chunks // NC

    def real_body(hc_hbm, src_hbm, dst_hbm, zeros_hbm, agg_hbm,
                  src_v, dst_v, rows2, gsem, ssem, acc):
        ci = lax.axis_index("c")
        s = lax.axis_index("s")
        pltpu.sync_copy(src_hbm.at[pl.ds(s * EPT, EPT)], src_v)
        pltpu.sync_copy(dst_hbm.at[pl.ds(s * EPT, EPT)], dst_v)
        for lc in range(cpc):
            cglob = ci * cpc + lc
            delta = ci * cpc * N if lc == 0 else N

            def off(i, carry):
                src_v[pl.ds(i * 16, 16)] = src_v[pl.ds(i * 16, 16)] + delta
                return carry

            lax.fori_loop(0, EPT // 16, off, 0)
            pltpu.sync_copy(zeros_hbm.at[pl.ds(s * RPT, RPT)],
                            acc.at[pl.ds(s * RPT, RPT)])
            plsc.subcore_barrier()

            pltpu.async_copy(hc_hbm.at[src_v.at[pl.ds(0, GA)]],
                             rows2.at[0], gsem)

            def grp(g, carry):
                b = lax.rem(g, 2)
                pltpu.make_async_copy(hc_hbm.at[src_v.at[pl.ds(g * GA, GA)]],
                                      rows2.at[b], gsem).wait()
                pltpu.async_copy(rows2.at[b],
                                 acc.at[dst_v.at[pl.ds(g * GA, GA)]], ssem,
                                 add=True)

                @pl.when(g >= 1)
                def _():
                    pltpu.make_async_copy(
                        rows2.at[1 - b],
                        acc.at[dst_v.at[pl.ds((g - 1) * GA, GA)]], ssem).wait()

                @pl.when(g + 1 < NFULL)
                def _():
                    pltpu.async_copy(
                        hc_hbm.at[src_v.at[pl.ds((g + 1) * GA, GA)]],
                        rows2.at[1 - b], gsem)

                return carry

            lax.fori_loop(0, NFULL, grp, 0)
            pltpu.make_async_copy(
                rows2.at[(NFULL - 1) % 2],
                acc.at[dst_v.at[pl.ds((NFULL - 1) * GA, GA)]], ssem).wait()
            pltpu.sync_copy(hc_hbm.at[src_v.at[pl.ds(NFULL * GA, AT)]],
                            rows2.at[0, pl.ds(0, AT)])
            pltpu.sync_copy(rows2.at[0, pl.ds(0, AT)],
                            acc.at[dst_v.at[pl.ds(NFULL * GA, AT)]], add=True)
            plsc.subcore_barrier()
            pltpu.sync_copy(acc.at[pl.ds(s * RPT, RPT)],
                            agg_hbm.at[cglob, pl.ds(s * RPT, RPT)])
            plsc.subcore_barrier()

    return pl.kernel(
        real_body,
        out_type=jax.ShapeDtypeStruct((n_chunks, NP, LANE), jnp.float32),
        mesh=_mesh(),
        scratch_types=[
            pltpu.VMEM((EPT,), jnp.int32),
            pltpu.VMEM((EPT,), jnp.int32),
            pltpu.VMEM((2, GA, LANE), jnp.float32),
            pltpu.SemaphoreType.DMA,
            pltpu.SemaphoreType.DMA,
            pltpu.VMEM_SHARED((NP, LANE), jnp.float32),
        ],
    )(hc_flat, src, dst, zeros128)


BN = 400


def _tc_mm1_scale(deg, x, W1):
    c1 = H1 // LANE

    def body(deg_ref, x_ref, w_ref, hp_ref, dinv_ref):
        d = deg_ref[...][:, :, 0]
        dinv = jnp.where(d > 0, lax.rsqrt(jnp.maximum(d, 1.0)), 0.0)
        xw = jnp.dot(x_ref[...], w_ref[...],
                     preferred_element_type=jnp.float32)
        hp_ref[...] = (xw * dinv[0][:, None])[None]
        dinv_ref[...] = dinv[:, :, None]

    return pl.pallas_call(
        body,
        grid=(N // BN, c1),
        in_specs=[
            pl.BlockSpec((2, BN, LANE), lambda nb, c: (0, nb, 0)),
            pl.BlockSpec((BN, D_IN), lambda nb, c: (nb, 0)),
            pl.BlockSpec((D_IN, LANE), lambda nb, c: (0, c)),
        ],
        out_specs=[
            pl.BlockSpec((1, BN, LANE), lambda nb, c: (c, nb, 0)),
            pl.BlockSpec((2, BN, 1), lambda nb, c: (0, nb, 0)),
        ],
        out_shape=[
            jax.ShapeDtypeStruct((c1, N, LANE), jnp.float32),
            jax.ShapeDtypeStruct((2, N, 1), jnp.float32),
        ],
    )(deg, x, W1)


def _tc_layer2(agg1, dinv, W2):
    c1 = H1 // LANE
    c2 = H2 // LANE

    def body(agg_ref, dinv_ref, w2_ref, out_ref):
        dout = dinv_ref[0]
        din = dinv_ref[1]
        acc = jnp.zeros((BN, H2), jnp.float32)
        for c in range(c1):
            h = jnp.maximum(agg_ref[c] * din, 0.0)
            acc = acc + jnp.dot(h, w2_ref[c * LANE:(c + 1) * LANE, :],
                                preferred_element_type=jnp.float32)
        h2p = acc * dout
        for c in range(c2):
            out_ref[c] = h2p[:, c * LANE:(c + 1) * LANE]

    return pl.pallas_call(
        body,
        grid=(N // BN,),
        in_specs=[
            pl.BlockSpec((c1, BN, LANE), lambda nb: (0, nb, 0)),
            pl.BlockSpec((2, BN, 1), lambda nb: (0, nb, 0)),
            pl.BlockSpec((H1, H2), lambda nb: (0, 0)),
        ],
        out_specs=pl.BlockSpec((c2, BN, LANE), lambda nb: (0, nb, 0)),
        out_shape=jax.ShapeDtypeStruct((c2, N, LANE), jnp.float32),
    )(agg1, dinv, W2)


def _tc_final(agg2, dinv):
    c2 = H2 // LANE

    def body(agg_ref, dinv_ref, out_ref):
        din = dinv_ref[1]
        for c in range(c2):
            out_ref[:, c * LANE:(c + 1) * LANE] = agg_ref[c] * din

    return pl.pallas_call(
        body,
        grid=(N // BN,),
        in_specs=[
            pl.BlockSpec((c2, BN, LANE), lambda nb: (0, nb, 0)),
            pl.BlockSpec((2, BN, 1), lambda nb: (0, nb, 0)),
        ],
        out_specs=pl.BlockSpec((BN, H2), lambda nb: (nb, 0)),
        out_shape=jax.ShapeDtypeStruct((N, H2), jnp.float32),
    )(agg2, dinv)


def kernel(x, edge_index, W1, W2):
    src = edge_index[0]
    dst = edge_index[1]
    ones128 = jnp.ones((G, LANE), jnp.float32)
    zeros128 = jnp.zeros((NP, LANE), jnp.float32)

    deg = _sc_degree(edge_index.reshape(-1), ones128, zeros128)
    hp, dinv = _tc_mm1_scale(deg, x, W1)
    agg1 = _sc_aggregate(hp.reshape(-1, LANE), src, dst, zeros128,
                         H1 // LANE)
    h2c = _tc_layer2(agg1, dinv, W2)
    agg2 = _sc_aggregate(h2c.reshape(-1, LANE), src, dst, zeros128,
                         H2 // LANE)
    return _tc_final(agg2, dinv)


# speedup vs baseline: 1.0667x; 1.0667x over previous
import functools

import jax
import jax.numpy as jnp
from jax import lax
from jax.experimental import pallas as pl
from jax.experimental.pallas import tpu as pltpu
from jax.experimental.pallas import tpu_sc as plsc

N = 10000
E = 160000
D_IN = 256
H1 = 512
H2 = 256

LANE = 128
NS = 16
NC = 2
EPT = E // NS
G = 312
NG = EPT // G
GT = EPT - NG * G
GA = 112
NFULL = EPT // GA
AT = EPT - NFULL * GA
NP = 10112
RPT = NP // NS

_mesh = lambda: plsc.VectorSubcoreMesh(
    core_axis_name="c", subcore_axis_name="s", num_cores=NC, num_subcores=NS)


def _sc_degree(eidx_flat, ones128, zeros128):
    def body(eidx_hbm, ones_hbm, zeros_hbm, deg_hbm, idx_v, ones_v, dsem, acc):
        ci = lax.axis_index("c")
        s = lax.axis_index("s")
        pltpu.sync_copy(eidx_hbm.at[pl.ds(ci * E + s * EPT, EPT)], idx_v)
        pltpu.sync_copy(ones_hbm, ones_v)
        pltpu.sync_copy(zeros_hbm.at[pl.ds(s * RPT, RPT)],
                        acc.at[pl.ds(s * RPT, RPT)])
        plsc.subcore_barrier()

        W = 8

        def grp(g, carry):
            pltpu.async_copy(ones_v, acc.at[idx_v.at[pl.ds(g * G, G)]], dsem,
                             add=True)

            @pl.when(g >= W)
            def _():
                pltpu.make_async_copy(
                    ones_v, acc.at[idx_v.at[pl.ds((g - W) * G, G)]], dsem
                ).wait()

            return carry

        lax.fori_loop(0, NG, grp, 0)
        pltpu.async_copy(ones_v.at[pl.ds(0, GT)],
                         acc.at[idx_v.at[pl.ds(NG * G, GT)]], dsem, add=True)

        def drain(g, carry):
            pltpu.make_async_copy(
                ones_v, acc.at[idx_v.at[pl.ds(g * G, G)]], dsem).wait()
            return carry

        lax.fori_loop(NG - W, NG, drain, 0)
        pltpu.make_async_copy(ones_v.at[pl.ds(0, GT)],
                              acc.at[idx_v.at[pl.ds(NG * G, GT)]], dsem).wait()
        plsc.subcore_barrier()
        pltpu.sync_copy(acc.at[pl.ds(s * RPT, RPT)],
                        deg_hbm.at[ci, pl.ds(s * RPT, RPT)])

    return pl.kernel(
        body,
        out_type=jax.ShapeDtypeStruct((NC, NP, LANE), jnp.float32),
        mesh=_mesh(),
        scratch_types=[
            pltpu.VMEM((EPT,), jnp.int32),
            pltpu.VMEM((G, LANE), jnp.float32),
            pltpu.SemaphoreType.DMA,
            pltpu.VMEM_SHARED((NP, LANE), jnp.float32),
        ],
    )(eidx_flat, ones128, zeros128)


def _sc_aggregate(hc_flat, src, dst, zeros128, n_chunks):
    cpc = n_chunks // NC

    def real_body(hc_hbm, src_hbm, dst_hbm, zeros_hbm, agg_hbm,
                  src_v, dst_v, rows2, gsem, ssem, acc):
        ci = lax.axis_index("c")
        s = lax.axis_index("s")
        pltpu.sync_copy(src_hbm.at[pl.ds(s * EPT, EPT)], src_v)
        pltpu.sync_copy(dst_hbm.at[pl.ds(s * EPT, EPT)], dst_v)
        for lc in range(cpc):
            cglob = ci * cpc + lc
            delta = ci * cpc * N if lc == 0 else N

            def off(i, carry):
                src_v[pl.ds(i * 16, 16)] = src_v[pl.ds(i * 16, 16)] + delta
                return carry

            lax.fori_loop(0, EPT // 16, off, 0)
            pltpu.sync_copy(zeros_hbm.at[pl.ds(s * RPT, RPT)],
                            acc.at[pl.ds(s * RPT, RPT)])
            plsc.subcore_barrier()

            pltpu.async_copy(hc_hbm.at[src_v.at[pl.ds(0, GA)]],
                             rows2.at[0], gsem)

            def grp(g, carry):
                b = lax.rem(g, 2)
                pltpu.make_async_copy(hc_hbm.at[src_v.at[pl.ds(g * GA, GA)]],
                                      rows2.at[b], gsem).wait()
                pltpu.async_copy(rows2.at[b],
                                 acc.at[dst_v.at[pl.ds(g * GA, GA)]], ssem,
                                 add=True)

                @pl.when(g >= 1)
                def _():
                    pltpu.make_async_copy(
                        rows2.at[1 - b],
                        acc.at[dst_v.at[pl.ds((g - 1) * GA, GA)]], ssem).wait()

                @pl.when(g + 1 < NFULL)
                def _():
                    pltpu.async_copy(
                        hc_hbm.at[src_v.at[pl.ds((g + 1) * GA, GA)]],
                        rows2.at[1 - b], gsem)

                return carry

            lax.fori_loop(0, NFULL, grp, 0)
            pltpu.make_async_copy(
                rows2.at[(NFULL - 1) % 2],
                acc.at[dst_v.at[pl.ds((NFULL - 1) * GA, GA)]], ssem).wait()
            pltpu.sync_copy(hc_hbm.at[src_v.at[pl.ds(NFULL * GA, AT)]],
                            rows2.at[0, pl.ds(0, AT)])
            pltpu.sync_copy(rows2.at[0, pl.ds(0, AT)],
                            acc.at[dst_v.at[pl.ds(NFULL * GA, AT)]], add=True)
            plsc.subcore_barrier()
            pltpu.sync_copy(acc.at[pl.ds(s * RPT, RPT)],
                            agg_hbm.at[cglob, pl.ds(s * RPT, RPT)])
            plsc.subcore_barrier()

    return pl.kernel(
        real_body,
        out_type=jax.ShapeDtypeStruct((n_chunks, NP, LANE), jnp.float32),
        mesh=_mesh(),
        scratch_types=[
            pltpu.VMEM((EPT,), jnp.int32),
            pltpu.VMEM((EPT,), jnp.int32),
            pltpu.VMEM((2, GA, LANE), jnp.float32),
            pltpu.SemaphoreType.DMA,
            pltpu.SemaphoreType.DMA,
            pltpu.VMEM_SHARED((NP, LANE), jnp.float32),
        ],
    )(hc_flat, src, dst, zeros128)


BN = 400


def _tc_matmul1(x, W1):
    c1 = H1 // LANE

    def body(x_ref, w_ref, out_ref):
        out_ref[...] = jnp.dot(x_ref[...], w_ref[...],
                               preferred_element_type=jnp.float32)[None]

    return pl.pallas_call(
        body,
        grid=(N // BN, c1),
        in_specs=[
            pl.BlockSpec((BN, D_IN), lambda nb, c: (nb, 0)),
            pl.BlockSpec((D_IN, LANE), lambda nb, c: (0, c)),
        ],
        out_specs=pl.BlockSpec((1, BN, LANE), lambda nb, c: (c, nb, 0)),
        out_shape=jax.ShapeDtypeStruct((c1, N, LANE), jnp.float32),
    )(x, W1)


def _tc_scale(deg, xwc):
    c1 = H1 // LANE

    def body(deg_ref, xw_ref, hp_ref, dinv_ref):
        d = deg_ref[...][:, :, 0]
        dinv = jnp.where(d > 0, lax.rsqrt(jnp.maximum(d, 1.0)), 0.0)
        hp_ref[...] = xw_ref[...] * dinv[0][None, :, None]
        dinv_ref[...] = dinv[:, :, None]

    return pl.pallas_call(
        body,
        grid=(N // BN,),
        in_specs=[
            pl.BlockSpec((2, BN, LANE), lambda nb: (0, nb, 0)),
            pl.BlockSpec((c1, BN, LANE), lambda nb: (0, nb, 0)),
        ],
        out_specs=[
            pl.BlockSpec((c1, BN, LANE), lambda nb: (0, nb, 0)),
            pl.BlockSpec((2, BN, 1), lambda nb: (0, nb, 0)),
        ],
        out_shape=[
            jax.ShapeDtypeStruct((c1, N, LANE), jnp.float32),
            jax.ShapeDtypeStruct((2, N, 1), jnp.float32),
        ],
    )(deg, xwc)


def _tc_layer2(agg1, dinv, W2):
    c1 = H1 // LANE
    c2 = H2 // LANE

    def body(agg_ref, dinv_ref, w2_ref, out_ref):
        dout = dinv_ref[0]
        din = dinv_ref[1]
        acc = jnp.zeros((BN, H2), jnp.float32)
        for c in range(c1):
            h = jnp.maximum(agg_ref[c] * din, 0.0)
            acc = acc + jnp.dot(h, w2_ref[c * LANE:(c + 1) * LANE, :],
                                preferred_element_type=jnp.float32)
        h2p = acc * dout
        for c in range(c2):
            out_ref[c] = h2p[:, c * LANE:(c + 1) * LANE]

    return pl.pallas_call(
        body,
        grid=(N // BN,),
        in_specs=[
            pl.BlockSpec((c1, BN, LANE), lambda nb: (0, nb, 0)),
            pl.BlockSpec((2, BN, 1), lambda nb: (0, nb, 0)),
            pl.BlockSpec((H1, H2), lambda nb: (0, 0)),
        ],
        out_specs=pl.BlockSpec((c2, BN, LANE), lambda nb: (0, nb, 0)),
        out_shape=jax.ShapeDtypeStruct((c2, N, LANE), jnp.float32),
    )(agg1, dinv, W2)


def _tc_final(agg2, dinv):
    c2 = H2 // LANE

    def body(agg_ref, dinv_ref, out_ref):
        din = dinv_ref[1]
        for c in range(c2):
            out_ref[:, c * LANE:(c + 1) * LANE] = agg_ref[c] * din

    return pl.pallas_call(
        body,
        grid=(N // BN,),
        in_specs=[
            pl.BlockSpec((c2, BN, LANE), lambda nb: (0, nb, 0)),
            pl.BlockSpec((2, BN, 1), lambda nb: (0, nb, 0)),
        ],
        out_specs=pl.BlockSpec((BN, H2), lambda nb: (nb, 0)),
        out_shape=jax.ShapeDtypeStruct((N, H2), jnp.float32),
    )(agg2, dinv)


def kernel(x, edge_index, W1, W2):
    src = edge_index[0]
    dst = edge_index[1]
    ones128 = jnp.ones((G, LANE), jnp.float32)
    zeros128 = jnp.zeros((NP, LANE), jnp.float32)

    deg = _sc_degree(edge_index.reshape(-1), ones128, zeros128)
    xwc = _tc_matmul1(x, W1)
    hp, dinv = _tc_scale(deg, xwc)
    agg1 = _sc_aggregate(hp.reshape(-1, LANE), src, dst, zeros128,
                         H1 // LANE)
    h2c = _tc_layer2(agg1, dinv, W2)
    agg2 = _sc_aggregate(h2c.reshape(-1, LANE), src, dst, zeros128,
                         H2 // LANE)
    return _tc_final(agg2, dinv)
